# baseline (speedup 1.0000x reference)
import functools

import jax
import jax.numpy as jnp
from jax import lax
from jax.experimental import pallas as pl
from jax.experimental.pallas import tpu as pltpu
from jax.experimental.pallas import tpu_sc as plsc

V = 50
D = 18432
BATCH = 128
PLEN = 50
NC = 2
NS = 16
NW = NC * NS
NSLICE = 4
BSLICE = BATCH // NSLICE
BPW = PLEN
BPAD = 64
GROUP = 16
NGROUP = BPW // GROUP
NREM = BPW - NGROUP * GROUP
NBUF = 2


def _gather_body(tbl, idxs, out, idx_vm, *rest):
    bufs = rest[:NBUF]
    gsem = rest[NBUF:2 * NBUF]
    psem = rest[2 * NBUF:3 * NBUF]

    s = lax.axis_index("s")
    wid = s * NC + lax.axis_index("c")
    pltpu.sync_copy(idxs.at[wid], idx_vm)

    def gather(iv, b):
        pltpu.async_copy(tbl.at[pl.ds(iv, 1)], bufs[b], gsem[b])

    def put(c, b):
        pltpu.async_copy(bufs[b], out.at[wid, pl.ds(c, 1)], psem[b])

    def wait_gather(b):
        pltpu.make_async_copy(tbl.at[pl.ds(0, 1)], bufs[b], gsem[b]).wait()

    def wait_put(b):
        pltpu.make_async_copy(bufs[b], out.at[0, pl.ds(0, 1)], psem[b]).wait()

    vec0 = idx_vm[pl.ds(0, GROUP)]
    for b in range(NBUF):
        gather(vec0[b], b)

    def body(i, carry):
        c0 = i * GROUP
        vec = idx_vm[pl.ds(pl.multiple_of(c0, GROUP), GROUP)]
        vec_next = idx_vm[pl.ds(pl.multiple_of(c0 + GROUP, GROUP), GROUP)]
        for j in range(GROUP):
            b = j % NBUF
            wait_gather(b)
            put(c0 + j, b)
            wait_put(b)
            nj = j + NBUF
            iv = vec[nj] if nj < GROUP else vec_next[nj - GROUP]
            gather(iv, b)
        return carry

    lax.fori_loop(0, NGROUP, body, 0)
    for j in range(NREM):
        c = NGROUP * GROUP + j
        b = j % NBUF
        wait_gather(b)
        put(c, b)
        wait_put(b)


@functools.cache
def _build_gather_kernel():
    mesh = plsc.VectorSubcoreMesh(
        core_axis_name="c", subcore_axis_name="s", num_cores=NC, num_subcores=NS
    )
    return functools.partial(
        pl.kernel,
        out_type=jax.ShapeDtypeStruct((BSLICE, PLEN, D), jnp.float32),
        mesh=mesh,
        scratch_types=[
            pltpu.VMEM((BPAD,), jnp.int32),
        ]
        + [pltpu.VMEM((1, D), jnp.float32)] * NBUF
        + [pltpu.SemaphoreType.DMA] * (2 * NBUF),
    )(_gather_body)


def kernel(prefix, embedding_table):
    idx = prefix.astype(jnp.int32)
    idx = jnp.pad(idx, ((0, 0), (0, BPAD - BPW)))
    k = _build_gather_kernel()
    parts = [
        k(embedding_table, idx[i * BSLICE:(i + 1) * BSLICE])
        for i in range(NSLICE)
    ]
    return jnp.concatenate(parts, axis=0)


# speedup vs baseline: 1.5885x; 1.5885x over previous
import functools

import jax
import jax.numpy as jnp
from jax import lax
from jax.experimental import pallas as pl
from jax.experimental.pallas import tpu as pltpu
from jax.experimental.pallas import tpu_sc as plsc

V = 50
D = 18432
BATCH = 128
PLEN = 50
B = BATCH * PLEN
NC = 2
NS = 16
NW = NC * NS
BPW = B // NW
BPAD = 224
GROUP = 16
NGROUP = BPW // GROUP
NREM = BPW - NGROUP * GROUP
NBUF = 4


def _gather_body(tbl, idxs, out, idx_vm, *rest):
    bufs = rest[:NBUF]
    gsem = rest[NBUF:2 * NBUF]
    psem = rest[2 * NBUF:3 * NBUF]

    s = lax.axis_index("s")
    wid = s * NC + lax.axis_index("c")
    base = wid * BPW
    pltpu.sync_copy(idxs.at[wid], idx_vm)

    def gather(iv, b):
        pltpu.async_copy(tbl.at[pl.ds(iv, 1)], bufs[b], gsem[b])

    def put(c, b):
        bb = c // PLEN
        pp = c - bb * PLEN
        pltpu.async_copy(bufs[b], out.at[4 * wid + bb, pl.ds(pp, 1)], psem[b])

    def wait_gather(b):
        pltpu.make_async_copy(tbl.at[pl.ds(0, 1)], bufs[b], gsem[b]).wait()

    def wait_put(b):
        pltpu.make_async_copy(bufs[b], out.at[0, pl.ds(0, 1)], psem[b]).wait()

    vec0 = idx_vm[pl.ds(0, GROUP)]
    for b in range(NBUF):
        gather(vec0[b], b)

    def body(i, carry):
        c0 = i * GROUP
        vec = idx_vm[pl.ds(pl.multiple_of(c0, GROUP), GROUP)]
        vec_next = idx_vm[pl.ds(pl.multiple_of(c0 + GROUP, GROUP), GROUP)]
        for j in range(GROUP):
            b = j % NBUF
            wait_gather(b)
            put(c0 + j, b)
            wait_put(b)
            nj = j + NBUF
            iv = vec[nj] if nj < GROUP else vec_next[nj - GROUP]
            gather(iv, b)
        return carry

    lax.fori_loop(0, NGROUP, body, 0)
    vec_r = idx_vm[pl.ds(NGROUP * GROUP, GROUP)]
    for j in range(NREM):
        c = NGROUP * GROUP + j
        b = j % NBUF
        wait_gather(b)
        put(c, b)
        wait_put(b)
        if j + NBUF < NREM:
            gather(vec_r[j + NBUF], b)


@functools.cache
def _build_gather_kernel():
    mesh = plsc.VectorSubcoreMesh(
        core_axis_name="c", subcore_axis_name="s", num_cores=NC, num_subcores=NS
    )
    return functools.partial(
        pl.kernel,
        out_type=jax.ShapeDtypeStruct((BATCH, PLEN, D), jnp.float32),
        mesh=mesh,
        compiler_params=pltpu.CompilerParams(use_tc_tiling_on_sc=True),
        scratch_types=[
            pltpu.VMEM((BPAD,), jnp.int32),
        ]
        + [pltpu.VMEM((1, D), jnp.float32)] * NBUF
        + [pltpu.SemaphoreType.DMA] * (2 * NBUF),
    )(_gather_body)


def kernel(prefix, embedding_table):
    idx = prefix.reshape(NW, BPW).astype(jnp.int32)
    idx = jnp.pad(idx, ((0, 0), (0, BPAD - BPW)))
    return _build_gather_kernel()(embedding_table, idx)
